# Initial kernel scaffold; baseline (speedup 1.0000x reference)
import jax
import jax.numpy as jnp
from jax.experimental import pallas as pl


def kernel(ftr, teachor_ftr, label, id_prototypes, idH):
    raise NotImplementedError("write your pallas kernel here")


# speedup vs baseline: 1.4325x; 1.4325x over previous
import jax
import jax.numpy as jnp
from jax.experimental import pallas as pl
from jax.experimental.pallas import tpu as pltpu

_K = 100
_Q = 0.03
_KP = 104


def _loss_body(gi_ref, diff_ref, out_ref):
    b = pl.program_id(0)

    @pl.when(b == 0)
    def _init():
        out_ref[0, 0] = 0.0

    gi = gi_ref[...]
    diff = diff_ref[...]
    dots = jax.lax.dot_general(
        gi, diff,
        dimension_numbers=(((2,), (1,)), ((0,), (0,))),
        preferred_element_type=jnp.float32,
    )
    contrib = jnp.maximum(dots - _Q, 0.0)
    out_ref[0, 0] += jnp.sum(contrib)


def kernel(ftr, teachor_ftr, label, id_prototypes, idH):
    B = ftr.shape[0]
    protos = id_prototypes.at[label].set(teachor_ftr)
    idx = idH[label, :_K]
    gi = protos[idx]
    gi = jnp.pad(gi, ((0, 0), (0, _KP - _K), (0, 0)))
    diff = ftr - teachor_ftr

    total = pl.pallas_call(
        _loss_body,
        grid=(B // 8,),
        in_specs=[
            pl.BlockSpec((8, _KP, 128), lambda b: (b, 0, 0)),
            pl.BlockSpec((8, 128), lambda b: (b, 0)),
        ],
        out_specs=pl.BlockSpec(memory_space=pltpu.SMEM),
        out_shape=jax.ShapeDtypeStruct((1, 1), jnp.float32),
    )(gi, diff)
    return total[0, 0] / (B * _K)


# speedup vs baseline: 1.8033x; 1.2588x over previous
import functools

import jax
import jax.numpy as jnp
from jax import lax
from jax.experimental import pallas as pl
from jax.experimental.pallas import tpu as pltpu
from jax.experimental.pallas import tpu_sc as plsc

_K = 100
_Q = 0.03
_NUM_IDS = 100000
_D = 128
_B = 1024
_OWNER_PAD = 100096
_NW = 32
_SPW = _B // _NW
_KP = 112


def _tc_body(ftr_ref, tb_ref, ta_ref, td_ref, diff_ref):
    diffb = ftr_ref[...] - tb_ref[...]
    diff_ref[...] = diffb
    td_ref[...] = lax.dot_general(
        diffb, ta_ref[...],
        dimension_numbers=(((1,), (1,)), ((), ())),
        preferred_element_type=jnp.float32,
    )


def _sidx_body(labcol_ref, lab2d_ref, sidx_ref):
    for a in range(8):
        col = labcol_ref[:, pl.ds(a, 1)]
        j_col = a * 128 + lax.broadcasted_iota(jnp.int32, (128, 128), 0)
        dup_acc = jnp.zeros((128, 128), jnp.bool_)
        for q in range(8):
            row = lab2d_ref[pl.ds(q, 1), :]
            jp = q * 128 + lax.broadcasted_iota(jnp.int32, (128, 128), 1)
            dup_acc = dup_acc | ((col == row) & (jp > j_col))
        dup = jnp.any(dup_acc, axis=1, keepdims=True)
        sidx_ref[:, pl.ds(a, 1)] = jnp.where(dup, _NUM_IDS, col)


def _tc_prep(ftr, teachor_ftr, label):
    label2d = label.reshape(8, 128)
    label_colT = label2d.T
    td, diff = pl.pallas_call(
        _tc_body,
        grid=(8,),
        in_specs=[
            pl.BlockSpec((128, _D), lambda g: (g, 0)),
            pl.BlockSpec((128, _D), lambda g: (g, 0)),
            pl.BlockSpec((_B, _D), lambda g: (0, 0)),
        ],
        out_specs=[
            pl.BlockSpec((128, _B), lambda g: (g, 0)),
            pl.BlockSpec((128, _D), lambda g: (g, 0)),
        ],
        out_shape=[
            jax.ShapeDtypeStruct((_B, _B), jnp.float32),
            jax.ShapeDtypeStruct((_B, _D), jnp.float32),
        ],
    )(ftr, teachor_ftr, teachor_ftr)
    sidx_colT = pl.pallas_call(
        _sidx_body,
        out_shape=jax.ShapeDtypeStruct((128, 8), jnp.int32),
    )(label_colT, label2d)
    return td, diff, sidx_colT


def _make_sc_kernel():
    mesh = plsc.VectorSubcoreMesh(core_axis_name="c", subcore_axis_name="s")

    @functools.partial(
        pl.kernel,
        out_type=(
            jax.ShapeDtypeStruct((_NW * 16,), jnp.float32),
            jax.ShapeDtypeStruct((_OWNER_PAD,), jnp.int32),
        ),
        mesh=mesh,
        scratch_types=[
            pltpu.VMEM((_SPW,), jnp.int32),
            pltpu.VMEM((64,), jnp.int32),
            pltpu.VMEM((64,), jnp.int32),
            pltpu.VMEM((_SPW * _KP,), jnp.int32),
            pltpu.VMEM((_SPW * _KP,), jnp.int32),
            pltpu.VMEM((2, 128), jnp.int32),
            pltpu.VMEM((2, 128), jnp.int32),
            pltpu.VMEM((2, 128), jnp.int32),
            pltpu.VMEM((2, 128), jnp.int32),
            pltpu.VMEM((2, 128), jnp.float32),
            pltpu.VMEM((2, 112), jnp.float32),
            pltpu.VMEM((_K, _D), jnp.float32),
            pltpu.VMEM((_K, _D), jnp.float32),
            pltpu.VMEM((_SPW, _D), jnp.float32),
            pltpu.VMEM((16,), jnp.float32),
            pltpu.SemaphoreType.DMA,
            pltpu.SemaphoreType.DMA,
            pltpu.SemaphoreType.DMA,
            pltpu.SemaphoreType.DMA,
            pltpu.SemaphoreType.DMA,
        ],
    )
    def sc_k(label_hbm, sidx_hbm, idh_hbm, protos_hbm, diff_hbm, tdflat_hbm,
             loss_hbm, owner_hbm,
             labels_v, sidx_v, jval_v, idxall_v, ixbuf_v, own_v,
             jc_v, tdix_v, lv_v, tdg_v, vals_v, rows0_v, rows1_v, diff_v,
             out_v, semA0, semA1, semB0, semB1, sem_setup):
        c = lax.axis_index("c")
        s = lax.axis_index("s")
        wid = s * 2 + c
        base = wid * _SPW
        iota16 = lax.iota(jnp.int32, 16)

        pltpu.sync_copy(sidx_hbm.at[pl.ds(s * 64, 64)], sidx_v)
        for q in range(4):
            jval_v[pl.ds(q * 16, 16)] = iota16 + (s * 64 + q * 16)
        pltpu.async_copy(jval_v, owner_hbm.at[sidx_v], semA0).wait()

        pltpu.sync_copy(label_hbm.at[pl.ds(base, _SPW)], labels_v)
        pltpu.sync_copy(diff_hbm.at[pl.ds(base, _SPW)], diff_v)
        for g in range(_SPW // 16):
            lab16 = labels_v[pl.ds(g * 16, 16)] * _K
            for l2 in range(16):
                i2 = g * 16 + l2
                lab = lab16[l2]
                for ci in range(7):
                    k0 = ci * 16
                    kvec = jnp.minimum(iota16 + k0, _K - 1)
                    ixbuf_v[pl.ds(i2 * _KP + k0, 16)] = kvec + lab
        pltpu.async_copy(idh_hbm.at[ixbuf_v], idxall_v, sem_setup).wait()

        plsc.subcore_barrier()

        semA = (semA0, semA1)
        semB = (semB0, semB1)
        rows = (rows0_v, rows1_v)

        def fire1(i, p):
            idxv = idxall_v.at[pl.ds(i * _KP, _K)]
            pltpu.async_copy(protos_hbm.at[idxv], rows[p], semA[p])
            pltpu.async_copy(owner_hbm.at[idxv], own_v.at[p, pl.ds(0, _K)], semA[p])

        def drain1_fire2(i, p):
            idxv = idxall_v.at[pl.ds(i * _KP, _K)]
            pltpu.make_async_copy(protos_hbm.at[idxv], rows[p], semA[p]).wait()
            pltpu.make_async_copy(owner_hbm.at[idxv], own_v.at[p, pl.ds(0, _K)],
                                  semA[p]).wait()
            rowoff = (base + i) * _B
            for ci in range(7):
                k0 = ci * 16
                jc16 = lax.bitwise_and(own_v[p, pl.ds(k0, 16)], _B - 1)
                jc_v[p, pl.ds(k0, 16)] = jc16
                tdix_v[p, pl.ds(k0, 16)] = jc16 + rowoff
            pltpu.async_copy(label_hbm.at[jc_v.at[p, pl.ds(0, 112)]],
                             lv_v.at[p, pl.ds(0, 112)], semB[p])
            pltpu.async_copy(tdflat_hbm.at[tdix_v.at[p, pl.ds(0, 112)]],
                             tdg_v.at[p, pl.ds(0, 112)], semB[p])

        _dnums = lax.GatherDimensionNumbers(
            offset_dims=(), collapsed_slice_dims=(0,), start_index_map=(0,))
        perms = [lax.bitwise_xor(iota16, sh).reshape(16, 1) for sh in (8, 4, 2, 1)]

        def _shuf(v, pm):
            return lax.gather(
                v, pm, _dnums, (1,),
                mode=lax.GatherScatterMode.PROMISE_IN_BOUNDS)

        def dots(i, p):
            rv = rows[p]
            dr = [diff_v[i, pl.ds(ch * 16, 16)] for ch in range(8)]
            for ci in range(7):
                k0 = ci * 16
                val16 = jnp.zeros((16,), jnp.float32)
                for l in range(min(16, _K - k0)):
                    k = k0 + l
                    acc16 = dr[0] * rv[k, pl.ds(0, 16)]
                    for ch in range(1, 8):
                        acc16 = acc16 + dr[ch] * rv[k, pl.ds(ch * 16, 16)]
                    s = acc16
                    for pm in perms:
                        s = s + _shuf(s, pm)
                    val16 = jnp.where(iota16 == l, jnp.full((16,), s[0]), val16)
                vals_v[p, pl.ds(k0, 16)] = val16

        def drain2_correct(i, p, loss16):
            pltpu.make_async_copy(label_hbm.at[jc_v.at[p, pl.ds(0, 112)]],
                                  lv_v.at[p, pl.ds(0, 112)], semB[p]).wait()
            pltpu.make_async_copy(tdflat_hbm.at[tdix_v.at[p, pl.ds(0, 112)]],
                                  tdg_v.at[p, pl.ds(0, 112)], semB[p]).wait()
            for ci in range(7):
                k0 = ci * 16
                nv = min(16, _K - k0)
                val16 = vals_v[p, pl.ds(k0, 16)]
                id16 = idxall_v[pl.ds(i * _KP + k0, 16)]
                lv16 = lv_v[p, pl.ds(k0, 16)]
                td16 = tdg_v[p, pl.ds(k0, 16)]
                corrected = jnp.where(lv16 == id16, td16, val16)
                contrib = jnp.maximum(corrected - _Q, 0.0)
                if nv < 16:
                    contrib = jnp.where(iota16 < nv, contrib, 0.0)
                loss16 = loss16 + contrib
            return loss16

        fire1(0, 0)
        fire1(1, 1)

        def pair(t, loss16):
            i0 = t * 2
            drain1_fire2(i0, 0)
            dots(i0, 0)
            loss16 = drain2_correct(i0, 0, loss16)

            @pl.when(t < _SPW // 2 - 1)
            def _():
                fire1(i0 + 2, 0)

            drain1_fire2(i0 + 1, 1)
            dots(i0 + 1, 1)
            loss16 = drain2_correct(i0 + 1, 1, loss16)

            @pl.when(t < _SPW // 2 - 1)
            def _():
                fire1(i0 + 3, 1)

            return loss16

        loss16 = lax.fori_loop(0, _SPW // 2, pair,
                               jnp.zeros((16,), jnp.float32))

        out_v[...] = loss16
        pltpu.sync_copy(out_v, loss_hbm.at[pl.ds(wid * 16, 16)])

    return sc_k


_sc_kernel = _make_sc_kernel()


def kernel(ftr, teachor_ftr, label, id_prototypes, idH):
    td, diff, sidx_colT = _tc_prep(ftr, teachor_ftr, label)
    sidx = sidx_colT.T.reshape(_B)
    loss_parts, _ = _sc_kernel(label, sidx, idH.reshape(_NUM_IDS * _K),
                               id_prototypes, diff, td.reshape(_B * _B))
    return jnp.sum(loss_parts) / (_B * _K)
